# baseline (speedup 1.0000x reference)
import jax
import jax.numpy as jnp
from jax import lax
from jax.experimental import pallas as pl
from jax.experimental.pallas import tpu as pltpu
from jax.experimental.pallas import tpu_sc as plsc

N = 10000
E = 160000
D = 256

NC = 2
NS = 16
L = 16
NW = NC * NS

RPT = 312
ACC_ROWS = 332
DUMP = 328
SHIFT = 14
SMASK = (1 << SHIFT) - 1
EB = 1600
NBLK = E // EB
K = 128
TRASH = EB + K
CCAP = TRASH + L


def _sc_spmm(x_hbm, pk_hbm, w_hbm, hi_hbm,
             pk0, pk1, w0, w1, cpk, cw, csub, cd, cwv, rows, acc,
             semA, semB, semG):
    c = lax.axis_index("c")
    s = lax.axis_index("s")
    wid = c * NS + s
    rbase = wid * RPT
    rhigh = jnp.where(wid == NW - 1, N, rbase + RPT)

    zeros16 = jnp.zeros((L,), jnp.float32)
    iota16 = lax.iota(jnp.int32, L)

    @plsc.parallel_loop(0, ACC_ROWS, unroll=2)
    def _zero_row(r):
        for t in range(D // L):
            acc[r, pl.ds(t * L, L)] = zeros16

    def _init_c(t, _):
        cpk[pl.ds(t * L, L)] = jnp.zeros((L,), jnp.int32)
        cw[pl.ds(t * L, L)] = zeros16
        return 0

    lax.fori_loop(0, CCAP // L, _init_c, 0)

    def _issue(qoff):
        for t in range(K // L):
            pkv = cpk[pl.ds(qoff + t * L, L)]
            csub[pl.ds(t * L, L)] = pkv & SMASK
            cd[pl.ds(t * L, L)] = (pkv >> SHIFT) - rbase
            cwv[pl.ds(t * L, L)] = cw[pl.ds(qoff + t * L, L)]
        pltpu.async_copy(x_hbm.at[csub], rows, semG)

    def _acc_pending(cnt):
        pltpu.make_async_copy(x_hbm.at[csub], rows, semG).wait()

        def _grp(g, _):
            d16 = cd[pl.ds(g * L, L)]
            w16 = cwv[pl.ds(g * L, L)]
            valid = (g * L + iota16) < cnt
            d16 = jnp.where(valid, d16, DUMP)
            for j in range(L):
                dj = d16[j]
                wj = w16[j]
                r = g * L + j

                @plsc.parallel_loop(0, D // L, unroll=D // L)
                def _t(t):
                    plsc.addupdate(acc.at[dj, pl.ds(t * L, L)],
                                   rows[r, pl.ds(t * L, L)] * wj)

            return 0

        lax.fori_loop(0, K // L, _grp, 0)

    def _stage(blk, pk_buf, w_buf, sem):
        base_e = blk * EB
        pltpu.async_copy(pk_hbm.at[pl.ds(base_e, EB)], pk_buf, sem)
        pltpu.async_copy(w_hbm.at[pl.ds(base_e, EB)], w_buf, sem)

    def _wait_stage(pk_buf, w_buf, sem):
        pltpu.make_async_copy(pk_hbm.at[pl.ds(0, EB)], pk_buf, sem).wait()
        pltpu.make_async_copy(w_hbm.at[pl.ds(0, EB)], w_buf, sem).wait()

    def _scan_block(pk_buf, w_buf, wp, pend):
        @plsc.parallel_loop(0, EB // L, unroll=4, carry=wp)
        def _scan(t, off):
            pkv = pk_buf[pl.ds(t * L, L)]
            dv = pkv >> SHIFT
            m = (dv >= rbase) & (dv < rhigh)
            cum = plsc.cumsum(m.astype(jnp.int32))
            pos = jnp.where(m, off + cum - 1, TRASH + iota16)
            plsc.store_scatter(cpk, [pos], pkv)
            plsc.store_scatter(cw, [pos], w_buf[pl.ds(t * L, L)])
            return off + cum[L - 1]

        wp = _scan

        nfull = wp // K

        def _drain(q, pend):
            @pl.when(pend == 1)
            def _():
                _acc_pending(K)

            _issue(q * K)
            return jnp.int32(1)

        pend = lax.fori_loop(0, nfull, _drain, pend)

        @pl.when(nfull > 0)
        def _():
            for t in range(K // L):
                pv = cpk[pl.ds(nfull * K + t * L, L)]
                wv = cw[pl.ds(nfull * K + t * L, L)]
                cpk[pl.ds(t * L, L)] = pv
                cw[pl.ds(t * L, L)] = wv

        return wp - nfull * K, pend

    _stage(0, pk0, w0, semA)

    def _pair(i, carry):
        wp, pend = carry
        blk0 = i * 2
        _wait_stage(pk0, w0, semA)
        _stage(blk0 + 1, pk1, w1, semB)
        wp, pend = _scan_block(pk0, w0, wp, pend)
        _wait_stage(pk1, w1, semB)

        @pl.when(blk0 + 2 < NBLK)
        def _():
            _stage(blk0 + 2, pk0, w0, semA)

        wp, pend = _scan_block(pk1, w1, wp, pend)
        return wp, pend

    wp, pend = lax.fori_loop(0, NBLK // 2, _pair,
                             (jnp.int32(0), jnp.int32(0)))

    @pl.when(pend == 1)
    def _():
        _acc_pending(K)

    _issue(0)
    _acc_pending(wp)

    @pl.when(wid < NW - 1)
    def _():
        pltpu.sync_copy(acc.at[pl.ds(0, RPT)],
                        hi_hbm.at[pl.ds(rbase, RPT)])

    @pl.when(wid == NW - 1)
    def _():
        nr = N - (NW - 1) * RPT
        pltpu.sync_copy(acc.at[pl.ds(0, nr)],
                        hi_hbm.at[pl.ds((NW - 1) * RPT, nr)])


@jax.jit
def _spmm(x, pk, w):
    mesh = plsc.VectorSubcoreMesh(core_axis_name="c", subcore_axis_name="s",
                                  num_cores=NC, num_subcores=NS)
    f = pl.kernel(
        _sc_spmm,
        out_type=jax.ShapeDtypeStruct((N, D), jnp.float32),
        mesh=mesh,
        compiler_params=pltpu.CompilerParams(needs_layout_passes=False),
        scratch_types=[
            pltpu.VMEM((EB,), jnp.int32),
            pltpu.VMEM((EB,), jnp.int32),
            pltpu.VMEM((EB,), jnp.float32),
            pltpu.VMEM((EB,), jnp.float32),
            pltpu.VMEM((CCAP,), jnp.int32),
            pltpu.VMEM((CCAP,), jnp.float32),
            pltpu.VMEM((K,), jnp.int32),
            pltpu.VMEM((K,), jnp.int32),
            pltpu.VMEM((K,), jnp.float32),
            pltpu.VMEM((K, D), jnp.float32),
            pltpu.VMEM((ACC_ROWS, D), jnp.float32),
            pltpu.SemaphoreType.DMA,
            pltpu.SemaphoreType.DMA,
            pltpu.SemaphoreType.DMA,
        ],
    )
    return f(x, pk, w)


def _tc_combine_body(sc_ref, hi_ref, h0_ref, x_ref, w_ref, out_ref):
    theta = sc_ref[0]
    alpha = sc_ref[1]
    sup = (1.0 - alpha) * hi_ref[...] + alpha * h0_ref[...]
    mm = jnp.dot(sup, w_ref[...], preferred_element_type=jnp.float32)
    out_ref[...] = theta * mm + (1.0 - theta) * sup + x_ref[...]


@jax.jit
def _combine(hi, h0, x, W, theta, alpha):
    scalars = jnp.stack([jnp.float32(theta), jnp.float32(alpha)])
    blk = 1000
    grid = N // blk
    return pl.pallas_call(
        _tc_combine_body,
        grid=(grid,),
        in_specs=[
            pl.BlockSpec(memory_space=pltpu.SMEM),
            pl.BlockSpec((blk, D), lambda i: (i, 0)),
            pl.BlockSpec((blk, D), lambda i: (i, 0)),
            pl.BlockSpec((blk, D), lambda i: (i, 0)),
            pl.BlockSpec((D, D), lambda i: (0, 0)),
        ],
        out_specs=pl.BlockSpec((blk, D), lambda i: (i, 0)),
        out_shape=jax.ShapeDtypeStruct((N, D), jnp.float32),
    )(scalars, hi, h0, x, W)


def kernel(input, edge_index, edge_weight, h0, W, lamda, alpha, l):
    theta = jnp.log(lamda / (l + 1) + 1.0)
    pk = (edge_index[1] << SHIFT) | edge_index[0]
    hi = _spmm(input, pk, edge_weight)
    return _combine(hi, h0, input, W, theta, alpha)


# speedup vs baseline: 1.1361x; 1.1361x over previous
import jax
import jax.numpy as jnp
from jax import lax
from jax.experimental import pallas as pl
from jax.experimental.pallas import tpu as pltpu
from jax.experimental.pallas import tpu_sc as plsc

N = 10000
E = 160000
D = 256

NC = 2
NS = 16
L = 16
NW = NC * NS

RPT = 312
ACC_ROWS = 332
DUMP = 328
SHIFT = 14
SMASK = (1 << SHIFT) - 1
EB = 2000
NBLK = E // EB
K = 80
TRASH = EB + K
CCAP = TRASH + L


def _sc_spmm(x_hbm, pk_hbm, w_hbm, hi_hbm,
             pk0, pk1, w0, w1, cpk, cw, csub, cd, cwv, rows, acc,
             semA, semB, semG):
    c = lax.axis_index("c")
    s = lax.axis_index("s")
    wid = c * NS + s
    rbase = wid * RPT
    rhigh = jnp.where(wid == NW - 1, N, rbase + RPT)

    zeros16 = jnp.zeros((L,), jnp.float32)
    iota16 = lax.iota(jnp.int32, L)

    @plsc.parallel_loop(0, ACC_ROWS, unroll=2)
    def _zero_row(r):
        for t in range(D // L):
            acc[r, pl.ds(t * L, L)] = zeros16

    def _init_c(t, _):
        cpk[pl.ds(t * L, L)] = jnp.zeros((L,), jnp.int32)
        cw[pl.ds(t * L, L)] = zeros16
        return 0

    lax.fori_loop(0, CCAP // L, _init_c, 0)

    def _issue(qoff):
        for t in range(K // L):
            pkv = cpk[pl.ds(qoff + t * L, L)]
            csub[pl.ds(t * L, L)] = pkv & SMASK
            cd[pl.ds(t * L, L)] = (pkv >> SHIFT) - rbase
            cwv[pl.ds(t * L, L)] = cw[pl.ds(qoff + t * L, L)]
        pltpu.async_copy(x_hbm.at[csub], rows, semG)

    def _acc_pending(cnt):
        pltpu.make_async_copy(x_hbm.at[csub], rows, semG).wait()

        def _grp(g, _):
            d16 = cd[pl.ds(g * L, L)]
            w16 = cwv[pl.ds(g * L, L)]
            valid = (g * L + iota16) < cnt
            d16 = jnp.where(valid, d16, DUMP)
            for j in range(L):
                dj = d16[j]
                wj = w16[j]
                r = g * L + j

                @plsc.parallel_loop(0, D // L, unroll=D // L)
                def _t(t):
                    plsc.addupdate(acc.at[dj, pl.ds(t * L, L)],
                                   rows[r, pl.ds(t * L, L)] * wj)

            return 0

        lax.fori_loop(0, K // L, _grp, 0)

    def _stage(blk, pk_buf, w_buf, sem):
        base_e = blk * EB
        pltpu.async_copy(pk_hbm.at[pl.ds(base_e, EB)], pk_buf, sem)
        pltpu.async_copy(w_hbm.at[pl.ds(base_e, EB)], w_buf, sem)

    def _wait_stage(pk_buf, w_buf, sem):
        pltpu.make_async_copy(pk_hbm.at[pl.ds(0, EB)], pk_buf, sem).wait()
        pltpu.make_async_copy(w_hbm.at[pl.ds(0, EB)], w_buf, sem).wait()

    def _scan_block(pk_buf, w_buf, wp, pend):
        @plsc.parallel_loop(0, EB // L, unroll=4, carry=wp)
        def _scan(t, off):
            pkv = pk_buf[pl.ds(t * L, L)]
            dv = pkv >> SHIFT
            m = (dv >= rbase) & (dv < rhigh)
            cum = plsc.cumsum(m.astype(jnp.int32))
            pos = jnp.where(m, off + cum - 1, TRASH + iota16)
            plsc.store_scatter(cpk, [pos], pkv)
            plsc.store_scatter(cw, [pos], w_buf[pl.ds(t * L, L)])
            return off + cum[L - 1]

        wp = _scan

        nfull = wp // K

        def _drain(q, pend):
            @pl.when(pend == 1)
            def _():
                _acc_pending(K)

            _issue(q * K)
            return jnp.int32(1)

        pend = lax.fori_loop(0, nfull, _drain, pend)

        @pl.when(nfull > 0)
        def _():
            for t in range(K // L):
                pv = cpk[pl.ds(nfull * K + t * L, L)]
                wv = cw[pl.ds(nfull * K + t * L, L)]
                cpk[pl.ds(t * L, L)] = pv
                cw[pl.ds(t * L, L)] = wv

        return wp - nfull * K, pend

    _stage(0, pk0, w0, semA)

    def _pair(i, carry):
        wp, pend = carry
        blk0 = i * 2
        _wait_stage(pk0, w0, semA)
        _stage(blk0 + 1, pk1, w1, semB)
        wp, pend = _scan_block(pk0, w0, wp, pend)
        _wait_stage(pk1, w1, semB)

        @pl.when(blk0 + 2 < NBLK)
        def _():
            _stage(blk0 + 2, pk0, w0, semA)

        wp, pend = _scan_block(pk1, w1, wp, pend)
        return wp, pend

    wp, pend = lax.fori_loop(0, NBLK // 2, _pair,
                             (jnp.int32(0), jnp.int32(0)))

    @pl.when(pend == 1)
    def _():
        _acc_pending(K)

    _issue(0)
    _acc_pending(wp)

    @pl.when(wid < NW - 1)
    def _():
        pltpu.sync_copy(acc.at[pl.ds(0, RPT)],
                        hi_hbm.at[pl.ds(rbase, RPT)])

    @pl.when(wid == NW - 1)
    def _():
        nr = N - (NW - 1) * RPT
        pltpu.sync_copy(acc.at[pl.ds(0, nr)],
                        hi_hbm.at[pl.ds((NW - 1) * RPT, nr)])


@jax.jit
def _spmm(x, pk, w):
    mesh = plsc.VectorSubcoreMesh(core_axis_name="c", subcore_axis_name="s",
                                  num_cores=NC, num_subcores=NS)
    f = pl.kernel(
        _sc_spmm,
        out_type=jax.ShapeDtypeStruct((N, D), jnp.float32),
        mesh=mesh,
        compiler_params=pltpu.CompilerParams(needs_layout_passes=False),
        scratch_types=[
            pltpu.VMEM((EB,), jnp.int32),
            pltpu.VMEM((EB,), jnp.int32),
            pltpu.VMEM((EB,), jnp.float32),
            pltpu.VMEM((EB,), jnp.float32),
            pltpu.VMEM((CCAP,), jnp.int32),
            pltpu.VMEM((CCAP,), jnp.float32),
            pltpu.VMEM((K,), jnp.int32),
            pltpu.VMEM((K,), jnp.int32),
            pltpu.VMEM((K,), jnp.float32),
            pltpu.VMEM((K, D), jnp.float32),
            pltpu.VMEM((ACC_ROWS, D), jnp.float32),
            pltpu.SemaphoreType.DMA,
            pltpu.SemaphoreType.DMA,
            pltpu.SemaphoreType.DMA,
        ],
    )
    return f(x, pk, w)


def _tc_combine_body(sc_ref, hi_ref, h0_ref, x_ref, w_ref, out_ref):
    theta = sc_ref[0]
    alpha = sc_ref[1]
    sup = (1.0 - alpha) * hi_ref[...] + alpha * h0_ref[...]
    mm = jnp.dot(sup, w_ref[...], preferred_element_type=jnp.float32)
    out_ref[...] = theta * mm + (1.0 - theta) * sup + x_ref[...]


@jax.jit
def _combine(hi, h0, x, W, theta, alpha):
    scalars = jnp.stack([jnp.float32(theta), jnp.float32(alpha)])
    blk = 2000
    grid = N // blk
    return pl.pallas_call(
        _tc_combine_body,
        grid=(grid,),
        in_specs=[
            pl.BlockSpec(memory_space=pltpu.SMEM),
            pl.BlockSpec((blk, D), lambda i: (i, 0)),
            pl.BlockSpec((blk, D), lambda i: (i, 0)),
            pl.BlockSpec((blk, D), lambda i: (i, 0)),
            pl.BlockSpec((D, D), lambda i: (0, 0)),
        ],
        out_specs=pl.BlockSpec((blk, D), lambda i: (i, 0)),
        out_shape=jax.ShapeDtypeStruct((N, D), jnp.float32),
    )(scalars, hi, h0, x, W)


def kernel(input, edge_index, edge_weight, h0, W, lamda, alpha, l):
    theta = jnp.log(lamda / (l + 1) + 1.0)
    pk = (edge_index[1] << SHIFT) | edge_index[0]
    hi = _spmm(input, pk, edge_weight)
    return _combine(hi, h0, input, W, theta, alpha)


# speedup vs baseline: 1.1973x; 1.0539x over previous
import jax
import jax.numpy as jnp
from jax import lax
from jax.experimental import pallas as pl
from jax.experimental.pallas import tpu as pltpu
from jax.experimental.pallas import tpu_sc as plsc

N = 10000
E = 160000
D = 256

NC = 2
NS = 16
L = 16
NW = NC * NS

RPT = 312
ACC_ROWS = 332
DUMP = 328
SHIFT = 14
SMASK = (1 << SHIFT) - 1
EB = 2000
NBLK = E // EB
K = 80
TRASH = EB + K
CCAP = TRASH + L


def _sc_spmm(x_hbm, pk_hbm, w_hbm, hi_hbm,
             pk0, pk1, w0, w1, cpk, cw, csub, cd, cwv, rows, acc,
             semA, semB, semG):
    c = lax.axis_index("c")
    s = lax.axis_index("s")
    wid = c * NS + s
    rbase = wid * RPT
    rhigh = jnp.where(wid == NW - 1, N, rbase + RPT)

    zeros16 = jnp.zeros((L,), jnp.float32)
    iota16 = lax.iota(jnp.int32, L)

    @plsc.parallel_loop(0, ACC_ROWS, unroll=2)
    def _zero_row(r):
        for t in range(D // L):
            acc[r, pl.ds(t * L, L)] = zeros16

    def _init_c(t, _):
        cpk[pl.ds(t * L, L)] = jnp.zeros((L,), jnp.int32)
        cw[pl.ds(t * L, L)] = zeros16
        return 0

    lax.fori_loop(0, CCAP // L, _init_c, 0)

    def _issue(qoff):
        for t in range(K // L):
            pkv = cpk[pl.ds(qoff + t * L, L)]
            csub[pl.ds(t * L, L)] = pkv & SMASK
            cd[pl.ds(t * L, L)] = (pkv >> SHIFT) - rbase
            cwv[pl.ds(t * L, L)] = cw[pl.ds(qoff + t * L, L)]
        pltpu.async_copy(x_hbm.at[csub], rows, semG)

    def _acc_pending(cnt):
        pltpu.make_async_copy(x_hbm.at[csub], rows, semG).wait()

        def _grp(g, _):
            d16 = cd[pl.ds(g * L, L)]
            w16 = cwv[pl.ds(g * L, L)]
            valid = (g * L + iota16) < cnt
            d16 = jnp.where(valid, d16, DUMP)
            for j in range(L):
                dj = d16[j]
                wj = w16[j]
                r = g * L + j

                @plsc.parallel_loop(0, D // L, unroll=D // L)
                def _t(t):
                    plsc.addupdate(acc.at[dj, pl.ds(t * L, L)],
                                   rows[r, pl.ds(t * L, L)] * wj)

            return 0

        lax.fori_loop(0, K // L, _grp, 0)

    def _stage(blk, pk_buf, w_buf, sem):
        base_e = blk * EB
        pltpu.async_copy(pk_hbm.at[pl.ds(base_e, EB)], pk_buf, sem)
        pltpu.async_copy(w_hbm.at[pl.ds(base_e, EB)], w_buf, sem)

    def _wait_stage(pk_buf, w_buf, sem):
        pltpu.make_async_copy(pk_hbm.at[pl.ds(0, EB)], pk_buf, sem).wait()
        pltpu.make_async_copy(w_hbm.at[pl.ds(0, EB)], w_buf, sem).wait()

    def _scan_block(pk_buf, w_buf, wp, pend):
        lane15 = jnp.full((L,), L - 1, jnp.int32)

        @plsc.parallel_loop(0, EB // L, unroll=4,
                            carry=jnp.zeros((L,), jnp.int32) + wp)
        def _scan(t, offv):
            pkv = pk_buf[pl.ds(t * L, L)]
            dv = pkv >> SHIFT
            m = (dv >= rbase) & (dv < rhigh)
            cum = plsc.cumsum(m.astype(jnp.int32))
            pos = jnp.where(m, offv + cum - 1, TRASH + iota16)
            plsc.store_scatter(cpk, [pos], pkv)
            plsc.store_scatter(cw, [pos], w_buf[pl.ds(t * L, L)])
            bcast = lax.gather(
                cum, lane15[:, None],
                lax.GatherDimensionNumbers(offset_dims=(),
                                           collapsed_slice_dims=(0,),
                                           start_index_map=(0,)),
                (1,), mode=lax.GatherScatterMode.PROMISE_IN_BOUNDS)
            return offv + bcast

        wp = _scan[0]

        nfull = wp // K

        def _drain(q, pend):
            @pl.when(pend == 1)
            def _():
                _acc_pending(K)

            _issue(q * K)
            return jnp.int32(1)

        pend = lax.fori_loop(0, nfull, _drain, pend)

        @pl.when(nfull > 0)
        def _():
            for t in range(K // L):
                pv = cpk[pl.ds(nfull * K + t * L, L)]
                wv = cw[pl.ds(nfull * K + t * L, L)]
                cpk[pl.ds(t * L, L)] = pv
                cw[pl.ds(t * L, L)] = wv

        return wp - nfull * K, pend

    _stage(0, pk0, w0, semA)

    def _pair(i, carry):
        wp, pend = carry
        blk0 = i * 2
        _wait_stage(pk0, w0, semA)
        _stage(blk0 + 1, pk1, w1, semB)
        wp, pend = _scan_block(pk0, w0, wp, pend)
        _wait_stage(pk1, w1, semB)

        @pl.when(blk0 + 2 < NBLK)
        def _():
            _stage(blk0 + 2, pk0, w0, semA)

        wp, pend = _scan_block(pk1, w1, wp, pend)
        return wp, pend

    wp, pend = lax.fori_loop(0, NBLK // 2, _pair,
                             (jnp.int32(0), jnp.int32(0)))

    @pl.when(pend == 1)
    def _():
        _acc_pending(K)

    _issue(0)
    _acc_pending(wp)

    @pl.when(wid < NW - 1)
    def _():
        pltpu.sync_copy(acc.at[pl.ds(0, RPT)],
                        hi_hbm.at[pl.ds(rbase, RPT)])

    @pl.when(wid == NW - 1)
    def _():
        nr = N - (NW - 1) * RPT
        pltpu.sync_copy(acc.at[pl.ds(0, nr)],
                        hi_hbm.at[pl.ds((NW - 1) * RPT, nr)])


@jax.jit
def _spmm(x, pk, w):
    mesh = plsc.VectorSubcoreMesh(core_axis_name="c", subcore_axis_name="s",
                                  num_cores=NC, num_subcores=NS)
    f = pl.kernel(
        _sc_spmm,
        out_type=jax.ShapeDtypeStruct((N, D), jnp.float32),
        mesh=mesh,
        compiler_params=pltpu.CompilerParams(needs_layout_passes=False),
        scratch_types=[
            pltpu.VMEM((EB,), jnp.int32),
            pltpu.VMEM((EB,), jnp.int32),
            pltpu.VMEM((EB,), jnp.float32),
            pltpu.VMEM((EB,), jnp.float32),
            pltpu.VMEM((CCAP,), jnp.int32),
            pltpu.VMEM((CCAP,), jnp.float32),
            pltpu.VMEM((K,), jnp.int32),
            pltpu.VMEM((K,), jnp.int32),
            pltpu.VMEM((K,), jnp.float32),
            pltpu.VMEM((K, D), jnp.float32),
            pltpu.VMEM((ACC_ROWS, D), jnp.float32),
            pltpu.SemaphoreType.DMA,
            pltpu.SemaphoreType.DMA,
            pltpu.SemaphoreType.DMA,
        ],
    )
    return f(x, pk, w)


def _tc_combine_body(sc_ref, hi_ref, h0_ref, x_ref, w_ref, out_ref):
    theta = sc_ref[0]
    alpha = sc_ref[1]
    sup = (1.0 - alpha) * hi_ref[...] + alpha * h0_ref[...]
    mm = jnp.dot(sup, w_ref[...], preferred_element_type=jnp.float32)
    out_ref[...] = theta * mm + (1.0 - theta) * sup + x_ref[...]


@jax.jit
def _combine(hi, h0, x, W, theta, alpha):
    scalars = jnp.stack([jnp.float32(theta), jnp.float32(alpha)])
    blk = 2000
    grid = N // blk
    return pl.pallas_call(
        _tc_combine_body,
        grid=(grid,),
        in_specs=[
            pl.BlockSpec(memory_space=pltpu.SMEM),
            pl.BlockSpec((blk, D), lambda i: (i, 0)),
            pl.BlockSpec((blk, D), lambda i: (i, 0)),
            pl.BlockSpec((blk, D), lambda i: (i, 0)),
            pl.BlockSpec((D, D), lambda i: (0, 0)),
        ],
        out_specs=pl.BlockSpec((blk, D), lambda i: (i, 0)),
        out_shape=jax.ShapeDtypeStruct((N, D), jnp.float32),
    )(scalars, hi, h0, x, W)


def kernel(input, edge_index, edge_weight, h0, W, lamda, alpha, l):
    theta = jnp.log(lamda / (l + 1) + 1.0)
    pk = (edge_index[1] << SHIFT) | edge_index[0]
    hi = _spmm(input, pk, edge_weight)
    return _combine(hi, h0, input, W, theta, alpha)
